# baseline (speedup 1.0000x reference)
import jax
import jax.numpy as jnp
from jax.experimental import pallas as pl
from jax.experimental.pallas import tpu as pltpu

B, K, L = 32, 128, 2048
SPB = 4
_ONE_BITS = 0x3F800000
_BIG_BITS = 0x7F000000
_ITERS = 26


def _body(mask_ref, rand_ref, ratio_ref, out_ref):
    g = pl.program_id(0)
    rfm = mask_ref[...] * rand_ref[...]
    bits = jax.lax.bitcast_convert_type(rfm, jnp.int32)
    ones = jnp.ones((L,), jnp.float32)

    ks = []
    for j in range(SPB):
        num_obs = jnp.sum(jnp.dot(mask_ref[j], ones))
        ks.append(jnp.floor(num_obs * ratio_ref[g * SPB + j] + jnp.float32(0.5)))

    def step(_, state):
        los, his = state
        new_los, new_his = [], []
        for j in range(SPB):
            lo, hi = los[j], his[j]
            mid = lo + (hi - lo) // 2
            c = jnp.count_nonzero(bits[j] >= mid).astype(jnp.float32)
            take = c >= ks[j]
            new_los.append(jnp.where(take, mid, lo))
            new_his.append(jnp.where(take, hi, mid))
        return tuple(new_los), tuple(new_his)

    init = (tuple(jnp.int32(0) for _ in range(SPB)),
            tuple(jnp.int32(_ONE_BITS) for _ in range(SPB)))
    los, _ = jax.lax.fori_loop(0, _ITERS, step, init)

    for j in range(SPB):
        thresh = jnp.where(ks[j] <= 0, jnp.int32(_BIG_BITS), los[j])
        keep = jnp.logical_and(bits[j] > 0, bits[j] < thresh)
        out_ref[j] = keep.astype(jnp.float32)


@jax.jit
def kernel(observed_mask, rand_vals, sample_ratios):
    return pl.pallas_call(
        _body,
        grid=(B // SPB,),
        in_specs=[
            pl.BlockSpec((SPB, K, L), lambda i: (i, 0, 0)),
            pl.BlockSpec((SPB, K, L), lambda i: (i, 0, 0)),
            pl.BlockSpec(memory_space=pltpu.SMEM),
        ],
        out_specs=pl.BlockSpec((SPB, K, L), lambda i: (i, 0, 0)),
        out_shape=jax.ShapeDtypeStruct((B, K, L), jnp.float32),
    )(observed_mask, rand_vals, sample_ratios)


# speedup vs baseline: 1.2827x; 1.2827x over previous
import jax
import jax.numpy as jnp
from jax.experimental import pallas as pl
from jax.experimental.pallas import tpu as pltpu

B, K, L = 32, 128, 2048
SPB = 8
_ONE_BITS = 0x3F800000
_BIG_BITS = 0x7F000000
_ITERS = 26


def _body(mask_ref, rand_ref, ratio_ref, out_ref):
    g = pl.program_id(0)
    rfm = mask_ref[...] * rand_ref[...]
    bits = jax.lax.bitcast_convert_type(rfm, jnp.int32)
    ones = jnp.ones((L,), jnp.float32)

    ks = []
    for j in range(SPB):
        num_obs = jnp.sum(jnp.dot(mask_ref[j], ones))
        ks.append(jnp.floor(num_obs * ratio_ref[g * SPB + j] + jnp.float32(0.5)))

    def step(_, state):
        los, his = state
        new_los, new_his = [], []
        for j in range(SPB):
            lo, hi = los[j], his[j]
            mid = lo + (hi - lo) // 2
            c = jnp.sum(jnp.dot((bits[j] >= mid).astype(jnp.float32), ones))
            take = c >= ks[j]
            new_los.append(jnp.where(take, mid, lo))
            new_his.append(jnp.where(take, hi, mid))
        return tuple(new_los), tuple(new_his)

    init = (tuple(jnp.int32(0) for _ in range(SPB)),
            tuple(jnp.int32(_ONE_BITS) for _ in range(SPB)))
    los, _ = jax.lax.fori_loop(0, _ITERS, step, init)

    for j in range(SPB):
        thresh = jnp.where(ks[j] <= 0, jnp.int32(_BIG_BITS), los[j])
        keep = jnp.logical_and(bits[j] > 0, bits[j] < thresh)
        out_ref[j] = keep.astype(jnp.float32)


@jax.jit
def kernel(observed_mask, rand_vals, sample_ratios):
    return pl.pallas_call(
        _body,
        grid=(B // SPB,),
        in_specs=[
            pl.BlockSpec((SPB, K, L), lambda i: (i, 0, 0)),
            pl.BlockSpec((SPB, K, L), lambda i: (i, 0, 0)),
            pl.BlockSpec(memory_space=pltpu.SMEM),
        ],
        out_specs=pl.BlockSpec((SPB, K, L), lambda i: (i, 0, 0)),
        out_shape=jax.ShapeDtypeStruct((B, K, L), jnp.float32),
    )(observed_mask, rand_vals, sample_ratios)
